# baseline (speedup 1.0000x reference)
import jax
import jax.numpy as jnp
import numpy as np
from jax.experimental import pallas as pl
from jax.experimental.pallas import tpu as pltpu

_B, _D, _N, _P = 1024, 128, 100000, 1000
_TEMP = 0.05
_NB = 5000
_NUM_BLK = _N // _NB
_SCALE = float(np.log2(np.e)) / _TEMP


def _norm_kernel(img_ref, nimg_ref):
    img = img_ref[...]
    norm = jnp.sqrt(jnp.sum(img * img, axis=1, keepdims=True))
    nimg_ref[...] = (img * _SCALE / jnp.maximum(norm, 1e-12)
                     ).astype(jnp.bfloat16)


def _sums_kernel(nimg_ref, pids_ref, feats_ref, mpids_ref, pos_ref, all_ref):
    i = pl.program_id(0)

    @pl.when(i == 0)
    def _init():
        pos_ref[...] = jnp.zeros_like(pos_ref)
        all_ref[...] = jnp.zeros_like(all_ref)

    feats = feats_ref[...].astype(jnp.bfloat16)
    scores = jax.lax.dot_general(
        nimg_ref[...], feats, (((1,), (1,)), ((), ())),
        preferred_element_type=jnp.float32)
    e = jnp.exp2(scores)
    labels = pids_ref[...] == mpids_ref[0]
    pos_ref[...] += jnp.sum(jnp.where(labels, e, 0.0), axis=1, keepdims=True)
    all_ref[...] += jnp.sum(e, axis=1, keepdims=True)


def _loss_kernel(pos_ref, all_ref, out_ref):
    loss = -jnp.log(pos_ref[...] / all_ref[...] + 1e-8)
    out_ref[...] = jnp.sum(loss).reshape(1, 1) / _B


def kernel(image_inputs, text_inputs, image_ids, pids, features, memory_pids):
    del text_inputs, image_ids
    nimg = pl.pallas_call(
        _norm_kernel,
        out_shape=jax.ShapeDtypeStruct((_B, _D), jnp.bfloat16),
    )(image_inputs)

    pids2 = pids.reshape(_B, 1)
    mpids3 = memory_pids.reshape(_NUM_BLK, 1, _NB)
    pos, all_ = pl.pallas_call(
        _sums_kernel,
        grid=(_NUM_BLK,),
        in_specs=[
            pl.BlockSpec((_B, _D), lambda i: (0, 0)),
            pl.BlockSpec((_B, 1), lambda i: (0, 0)),
            pl.BlockSpec((_NB, _D), lambda i: (i, 0)),
            pl.BlockSpec((1, 1, _NB), lambda i: (i, 0, 0)),
        ],
        out_specs=[
            pl.BlockSpec((_B, 1), lambda i: (0, 0)),
            pl.BlockSpec((_B, 1), lambda i: (0, 0)),
        ],
        out_shape=[
            jax.ShapeDtypeStruct((_B, 1), jnp.float32),
            jax.ShapeDtypeStruct((_B, 1), jnp.float32),
        ],
        compiler_params=pltpu.CompilerParams(
            dimension_semantics=("arbitrary",)),
    )(nimg, pids2, features, mpids3)

    out = pl.pallas_call(
        _loss_kernel,
        out_shape=jax.ShapeDtypeStruct((1, 1), jnp.float32),
    )(pos, all_)
    return out[0, 0]
